# Initial kernel scaffold; baseline (speedup 1.0000x reference)
import jax
import jax.numpy as jnp
from jax.experimental import pallas as pl


def kernel(x, edge_index, batch, x_cell_mut, edge_feat, params):
    raise NotImplementedError("write your pallas kernel here")


# speedup vs baseline: 2.4487x; 2.4487x over previous
import functools

import jax
import jax.numpy as jnp
from jax import lax
from jax.experimental import pallas as pl
from jax.experimental.pallas import tpu as pltpu
from jax.experimental.pallas import tpu_sc as plsc

NN = 10000
NB = 32
DIM = 32
E = 160000
EPAD = 163840
CH = 128
NCHUNK = EPAD // (32 * CH)
ACC_ROWS = 10240


FPAD = 384
FH = FPAD // 2
ACC_ROWS_X = 10112
CHX = 40


def _edge_agg_x(xf, src2, dst2, z192):
    mesh = plsc.VectorSubcoreMesh(core_axis_name="c", subcore_axis_name="s")
    epr = EPAD // 16
    nch = epr // CHX
    sl = ACC_ROWS_X // 16

    @functools.partial(
        pl.kernel,
        out_type=jax.ShapeDtypeStruct((2 * ACC_ROWS_X, FH), jnp.float32),
        mesh=mesh,
        compiler_params=pltpu.CompilerParams(use_tc_tiling_on_sc=False),
        scratch_types=[
            pltpu.VMEM_SHARED((ACC_ROWS_X, FH), jnp.float32),
            pltpu.VMEM((CHX,), jnp.int32),
            pltpu.VMEM((CHX,), jnp.int32),
            pltpu.VMEM((CHX, FH), jnp.float32),
            pltpu.SemaphoreType.DMA,
        ],
    )
    def k(x_hbm, src_hbm, dst_hbm, z_hbm, out_hbm, acc, sidx, didx, rows, sem):
        c = lax.axis_index("c")
        s = lax.axis_index("s")
        pltpu.sync_copy(z_hbm.at[pl.ds(s * sl, sl)], acc.at[pl.ds(s * sl, sl)])
        plsc.subcore_barrier()

        ebase = c * EPAD + s * epr

        def body(g, carry):
            off = ebase + g * CHX
            pltpu.sync_copy(src_hbm.at[pl.ds(off, CHX)], sidx)
            pltpu.sync_copy(dst_hbm.at[pl.ds(off, CHX)], didx)
            pltpu.async_copy(x_hbm.at[sidx], rows, sem).wait()
            pltpu.sync_copy(rows, acc.at[didx], add=True)
            return carry

        lax.fori_loop(0, nch, body, 0)
        plsc.subcore_barrier()
        pltpu.sync_copy(acc.at[pl.ds(s * sl, sl)],
                        out_hbm.at[pl.ds(c * ACC_ROWS_X + s * sl, sl)])

    return k(xf, src2, dst2, z192)


def _edge_agg(p, src, dst, zrows):
    mesh = plsc.VectorSubcoreMesh(core_axis_name="c", subcore_axis_name="s")

    @functools.partial(
        pl.kernel,
        out_type=jax.ShapeDtypeStruct((2 * ACC_ROWS, DIM), jnp.float32),
        mesh=mesh,
        compiler_params=pltpu.CompilerParams(use_tc_tiling_on_sc=False),
        scratch_types=[
            pltpu.VMEM_SHARED((ACC_ROWS, DIM), jnp.float32),
            pltpu.VMEM((CH,), jnp.int32),
            pltpu.VMEM((CH,), jnp.int32),
            pltpu.VMEM((CH, DIM), jnp.float32),
            pltpu.SemaphoreType.DMA,
        ],
    )
    def k(p_hbm, src_hbm, dst_hbm, z_hbm, out_hbm, acc, sidx, didx, rows, sem):
        c = lax.axis_index("c")
        s = lax.axis_index("s")
        wid = c * 16 + s
        zr = ACC_ROWS // 16
        pltpu.sync_copy(z_hbm.at[pl.ds(s * zr, zr)], acc.at[pl.ds(s * zr, zr)])
        plsc.subcore_barrier()

        ebase = wid * (NCHUNK * CH)

        def body(g, carry):
            off = ebase + g * CH
            pltpu.sync_copy(src_hbm.at[pl.ds(off, CH)], sidx)
            pltpu.sync_copy(dst_hbm.at[pl.ds(off, CH)], didx)
            pltpu.async_copy(p_hbm.at[sidx], rows, sem).wait()
            pltpu.sync_copy(rows, acc.at[didx], add=True)
            return carry

        lax.fori_loop(0, NCHUNK, body, 0)
        plsc.subcore_barrier()
        pltpu.sync_copy(acc.at[pl.ds(s * zr, zr)],
                        out_hbm.at[pl.ds(c * ACC_ROWS + s * zr, zr)])

    return k(p, src, dst, zrows)


def _gin_tail(s, w1_ref, b1_ref, w2_ref, b2_ref, g_ref, be_ref):
    z = jnp.maximum(jnp.dot(s, w1_ref[...],
                            preferred_element_type=jnp.float32) + b1_ref[...], 0.0)
    t = jnp.dot(z, w2_ref[...], preferred_element_type=jnp.float32) + b2_ref[...]
    hh = jnp.maximum(t, 0.0)
    mu = jnp.mean(hh, axis=0, keepdims=True)
    var = jnp.mean((hh - mu) * (hh - mu), axis=0, keepdims=True)
    return (hh - mu) / jnp.sqrt(var + 1e-5) * g_ref[...] + be_ref[...]


def _gin1(xp, aggx, w1p, b1, w2, b2, g, be):
    def body(x_ref, agg_ref, w1_ref, b1_ref, w2_ref, b2_ref, g_ref, be_ref, o_ref):
        s = x_ref[...] + jnp.concatenate(
            [agg_ref[:NN], agg_ref[ACC_ROWS_X:ACC_ROWS_X + NN]], axis=1)
        o_ref[...] = _gin_tail(s, w1_ref, b1_ref, w2_ref, b2_ref, g_ref, be_ref)

    return pl.pallas_call(
        body,
        out_shape=jax.ShapeDtypeStruct((NN, DIM), jnp.float32),
    )(xp, aggx, w1p, b1, w2, b2, g, be)


def _gin_mid(h, agg, w1, b1, w2, b2, g, be):
    def body(h_ref, agg_ref, w1_ref, b1_ref, w2_ref, b2_ref, g_ref, be_ref, o_ref):
        s = h_ref[...] + agg_ref[:NN] + agg_ref[ACC_ROWS:ACC_ROWS + NN]
        o_ref[...] = _gin_tail(s, w1_ref, b1_ref, w2_ref, b2_ref, g_ref, be_ref)

    return pl.pallas_call(
        body,
        out_shape=jax.ShapeDtypeStruct((NN, DIM), jnp.float32),
    )(h, agg, w1, b1, w2, b2, g, be)


def _gin_last(h, agg, w1, b1, w2, b2, g, be, batch2d, wxd, bxd):
    def body(h_ref, agg_ref, w1_ref, b1_ref, w2_ref, b2_ref, g_ref, be_ref,
             bt_ref, wxd_ref, bxd_ref, o_ref):
        s = h_ref[...] + agg_ref[:NN] + agg_ref[ACC_ROWS:ACC_ROWS + NN]
        hn = _gin_tail(s, w1_ref, b1_ref, w2_ref, b2_ref, g_ref, be_ref)
        seg = jax.lax.broadcasted_iota(jnp.int32, (NN, NB), 1)
        onehot = (bt_ref[...] == seg).astype(jnp.float32)
        pooled = jax.lax.dot_general(onehot, hn, (((0,), (0,)), ((), ())),
                                     preferred_element_type=jnp.float32,
                                     precision=jax.lax.Precision.HIGHEST)
        o_ref[...] = jnp.maximum(
            jnp.dot(pooled, wxd_ref[...], preferred_element_type=jnp.float32)
            + bxd_ref[...], 0.0)

    return pl.pallas_call(
        body,
        out_shape=jax.ShapeDtypeStruct((NB, 128), jnp.float32),
    )(h, agg, w1, b1, w2, b2, g, be, batch2d, wxd, bxd)


def _cnn(xc, w1c, b1c, w2f, b2c, w3f, b3c):
    T1, T2, T3 = 13125, 4368, 1449
    P1, P2, P3 = 4375, 1456, 483

    def max3(ref):
        return jnp.maximum(jnp.maximum(ref[0::3, :], ref[1::3, :]), ref[2::3, :])

    def body(x_ref, w1_ref, b1_ref, w2_ref, b2_ref, w3_ref, b3_ref, o_ref,
             s1, s2, s3):
        xb = x_ref[0]
        u1t = jnp.concatenate([xb[:, kk:kk + T1] for kk in range(8)], axis=0)
        u1 = jnp.transpose(u1t)
        o1 = jnp.dot(u1, w1_ref[...], preferred_element_type=jnp.float32)
        s1[...] = jnp.maximum(o1 + b1_ref[...], 0.0)
        h1 = max3(s1)

        u2 = jnp.concatenate([h1[kk:kk + T2, :] for kk in range(8)], axis=1)
        o2 = jnp.dot(u2, w2_ref[...], preferred_element_type=jnp.float32)
        s2[...] = jnp.maximum(o2 + b2_ref[...], 0.0)
        h2 = max3(s2)

        u3 = jnp.concatenate([h2[kk:kk + T3, :] for kk in range(8)], axis=1)
        o3 = jnp.dot(u3, w3_ref[...], preferred_element_type=jnp.float32)
        s3[...] = jnp.maximum(o3 + b3_ref[...], 0.0)
        h3 = max3(s3)

        o_ref[...] = jnp.transpose(h3).reshape(1, 128, P3)

    out = pl.pallas_call(
        body,
        grid=(NB,),
        scratch_shapes=[pltpu.VMEM((T1, 32), jnp.float32),
                        pltpu.VMEM((T2, 64), jnp.float32),
                        pltpu.VMEM((T3, 128), jnp.float32)],
        in_specs=[pl.BlockSpec((1, 1, 13132), lambda i: (i, 0, 0)),
                  pl.BlockSpec((8, 32), lambda i: (0, 0)),
                  pl.BlockSpec((1, 32), lambda i: (0, 0)),
                  pl.BlockSpec((256, 64), lambda i: (0, 0)),
                  pl.BlockSpec((1, 64), lambda i: (0, 0)),
                  pl.BlockSpec((512, 128), lambda i: (0, 0)),
                  pl.BlockSpec((1, 128), lambda i: (0, 0))],
        out_specs=pl.BlockSpec((1, 128, P3), lambda i: (i, 0, 0)),
        out_shape=jax.ShapeDtypeStruct((NB, 128, P3), jnp.float32),
    )(xc.reshape(NB, 1, 13132), w1c, b1c, w2f, b2c, w3f, b3c)
    return out.reshape(NB, 61824)


def _fc_xt(ct, w, b):
    KB = 8832

    def body(a_ref, w_ref, b_ref, o_ref):
        @pl.when(pl.program_id(0) == 0)
        def _():
            o_ref[...] = jnp.broadcast_to(b_ref[...], (NB, 128))
        o_ref[...] += jnp.dot(a_ref[...], w_ref[...],
                              preferred_element_type=jnp.float32)

    return pl.pallas_call(
        body,
        grid=(61824 // KB,),
        in_specs=[pl.BlockSpec((NB, KB), lambda k: (0, k)),
                  pl.BlockSpec((KB, 128), lambda k: (k, 0)),
                  pl.BlockSpec((1, 128), lambda k: (0, 0))],
        out_specs=pl.BlockSpec((NB, 128), lambda k: (0, 0)),
        out_shape=jax.ShapeDtypeStruct((NB, 128), jnp.float32),
    )(ct, w, b)


def _head(xg, ct, w1, b1, w2, b2, wo, bo):
    def body(xg_ref, ct_ref, w1_ref, b1_ref, w2_ref, b2_ref, wo_ref, bo_ref, o_ref):
        xc = jnp.concatenate([xg_ref[...], ct_ref[...]], axis=1)
        h = jnp.maximum(jnp.dot(xc, w1_ref[...],
                                preferred_element_type=jnp.float32) + b1_ref[...], 0.0)
        h = jnp.maximum(jnp.dot(h, w2_ref[...],
                                preferred_element_type=jnp.float32) + b2_ref[...], 0.0)
        o_ref[...] = jax.nn.sigmoid(
            jnp.dot(h, wo_ref[...], preferred_element_type=jnp.float32) + bo_ref[...])

    return pl.pallas_call(
        body,
        out_shape=jax.ShapeDtypeStruct((NB, 1), jnp.float32),
    )(xg, ct, w1, b1, w2, b2, wo, bo)


def kernel(x, edge_index, batch, x_cell_mut, edge_feat, params):
    gin = params['gin']
    bn = params['bn']

    src = edge_index[0].astype(jnp.int32)
    dst = edge_index[1].astype(jnp.int32)
    npad = EPAD - E
    src_p = jnp.concatenate([src, jnp.zeros((npad,), jnp.int32)])
    dst_p = jnp.concatenate([dst, jnp.full((npad,), NN, jnp.int32)])
    src2 = jnp.concatenate([src_p, src_p + NN])
    dst2 = jnp.concatenate([dst_p, dst_p])
    zrows = jnp.zeros((ACC_ROWS, DIM), jnp.float32)
    z192 = jnp.zeros((ACC_ROWS_X, FH), jnp.float32)

    def r2(v):
        return v.reshape(1, -1)

    xp = jnp.concatenate([x, jnp.zeros((NN, FPAD - x.shape[1]), jnp.float32)], axis=1)
    xf = jnp.concatenate([xp[:, :FH], xp[:, FH:]], axis=0)
    w1p = jnp.concatenate(
        [gin[0][0], jnp.zeros((FPAD - x.shape[1], DIM), jnp.float32)], axis=0)
    aggx = _edge_agg_x(xf, src2, dst2, z192)
    (w1, b1, w2, b2) = gin[0]
    (g, be) = bn[0]
    h = _gin1(xp, aggx, w1p, r2(b1), w2, r2(b2), r2(g), r2(be))
    for l in range(1, 4):
        agg = _edge_agg(h, src_p, dst_p, zrows)
        (w1, b1, w2, b2) = gin[l]
        (g, be) = bn[l]
        h = _gin_mid(h, agg, w1, r2(b1), w2, r2(b2), r2(g), r2(be))
    agg = _edge_agg(h, src_p, dst_p, zrows)
    (w1, b1, w2, b2) = gin[4]
    (g, be) = bn[4]
    wxd, bxd = params['fc1_xd']
    xg = _gin_last(h, agg, w1, r2(b1), w2, r2(b2), r2(g), r2(be),
                   batch.reshape(NN, 1).astype(jnp.int32), wxd, r2(bxd))

    (cw1, cb1), (cw2, cb2), (cw3, cb3) = params['conv_xt']
    w1c = cw1.reshape(32, 8).T
    w2f = cw2.transpose(2, 1, 0).reshape(256, 64)
    w3f = cw3.transpose(2, 1, 0).reshape(512, 128)
    ctf = _cnn(x_cell_mut.reshape(NB, 13132), w1c, cb1.reshape(1, 32),
               w2f, cb2.reshape(1, 64), w3f, cb3.reshape(1, 128))
    wxt, bxt = params['fc1_xt']
    ct = _fc_xt(ctf, wxt, r2(bxt))

    w1h, b1h = params['fc1']
    w2h, b2h = params['fc2']
    wo, bo = params['out']
    return _head(xg, ct, w1h, r2(b1h), w2h, r2(b2h), wo, r2(bo))


# speedup vs baseline: 2.8495x; 1.1637x over previous
import functools

import jax
import jax.numpy as jnp
from jax import lax
from jax.experimental import pallas as pl
from jax.experimental.pallas import tpu as pltpu
from jax.experimental.pallas import tpu_sc as plsc

NN = 10000
NB = 32
DIM = 32
E = 160000
EPAD = 163840
CH = 128
NCHUNK = EPAD // (32 * CH)
ACC_ROWS = 10240


FPAD = 352
FH = FPAD // 2
ACC_ROWS_X = 10112
CHX = 40


def _edge_agg_x(xf, src2, dst2, z192):
    mesh = plsc.VectorSubcoreMesh(core_axis_name="c", subcore_axis_name="s")
    epr = EPAD // 16
    nch = epr // CHX
    sl = ACC_ROWS_X // 16

    @functools.partial(
        pl.kernel,
        out_type=jax.ShapeDtypeStruct((2 * ACC_ROWS_X, FH), jnp.float32),
        mesh=mesh,
        compiler_params=pltpu.CompilerParams(use_tc_tiling_on_sc=False),
        scratch_types=[
            pltpu.VMEM_SHARED((ACC_ROWS_X, FH), jnp.float32),
            pltpu.VMEM((CHX,), jnp.int32), pltpu.VMEM((CHX,), jnp.int32),
            pltpu.VMEM((CHX,), jnp.int32), pltpu.VMEM((CHX,), jnp.int32),
            pltpu.VMEM((CHX, FH), jnp.float32), pltpu.VMEM((CHX, FH), jnp.float32),
            pltpu.SemaphoreType.DMA, pltpu.SemaphoreType.DMA,
            pltpu.SemaphoreType.DMA, pltpu.SemaphoreType.DMA,
        ],
    )
    def k(x_hbm, src_hbm, dst_hbm, z_hbm, out_hbm, acc,
          sidx0, sidx1, didx0, didx1, rows0, rows1, gs0, gs1, ss0, ss1):
        c = lax.axis_index("c")
        s = lax.axis_index("s")
        pltpu.sync_copy(z_hbm.at[pl.ds(s * sl, sl)], acc.at[pl.ds(s * sl, sl)])
        plsc.subcore_barrier()

        ebase = c * EPAD + s * epr

        def body(i, carry):
            c0 = ebase + (2 * i) * CHX
            c1 = c0 + CHX
            pltpu.sync_copy(src_hbm.at[pl.ds(c0, CHX)], sidx0)
            pltpu.sync_copy(dst_hbm.at[pl.ds(c0, CHX)], didx0)
            g0 = pltpu.async_copy(x_hbm.at[sidx0], rows0, gs0)
            pltpu.sync_copy(src_hbm.at[pl.ds(c1, CHX)], sidx1)
            pltpu.sync_copy(dst_hbm.at[pl.ds(c1, CHX)], didx1)
            g1 = pltpu.async_copy(x_hbm.at[sidx1], rows1, gs1)
            g0.wait()
            s0 = pltpu.async_copy(rows0, acc.at[didx0], ss0, add=True)
            g1.wait()
            s1 = pltpu.async_copy(rows1, acc.at[didx1], ss1, add=True)
            s0.wait()
            s1.wait()
            return carry

        lax.fori_loop(0, nch // 2, body, 0)
        plsc.subcore_barrier()
        pltpu.sync_copy(acc.at[pl.ds(s * sl, sl)],
                        out_hbm.at[pl.ds(c * ACC_ROWS_X + s * sl, sl)])

    return k(xf, src2, dst2, z192)


def _edge_agg(p, src, dst, zrows):
    mesh = plsc.VectorSubcoreMesh(core_axis_name="c", subcore_axis_name="s")

    NS = 4

    @functools.partial(
        pl.kernel,
        out_type=jax.ShapeDtypeStruct((2 * ACC_ROWS, DIM), jnp.float32),
        mesh=mesh,
        compiler_params=pltpu.CompilerParams(use_tc_tiling_on_sc=False),
        scratch_types=[
            pltpu.VMEM_SHARED((ACC_ROWS, DIM), jnp.float32),
            [pltpu.VMEM((CH,), jnp.int32)] * NS,
            [pltpu.VMEM((CH,), jnp.int32)] * NS,
            [pltpu.VMEM((CH, DIM), jnp.float32)] * NS,
            [pltpu.SemaphoreType.DMA] * NS,
            [pltpu.SemaphoreType.DMA] * NS,
        ],
    )
    def k(p_hbm, src_hbm, dst_hbm, z_hbm, out_hbm, acc, sidx, didx, rows, gs, ss):
        c = lax.axis_index("c")
        s = lax.axis_index("s")
        wid = c * 16 + s
        zr = ACC_ROWS // 16
        pltpu.sync_copy(z_hbm.at[pl.ds(s * zr, zr)], acc.at[pl.ds(s * zr, zr)])
        plsc.subcore_barrier()

        ebase = wid * (NCHUNK * CH)

        def body(i, carry):
            base = ebase + (NS * i) * CH
            gh = []
            for t in range(NS):
                off = base + t * CH
                pltpu.sync_copy(src_hbm.at[pl.ds(off, CH)], sidx[t])
                pltpu.sync_copy(dst_hbm.at[pl.ds(off, CH)], didx[t])
                gh.append(pltpu.async_copy(p_hbm.at[sidx[t]], rows[t], gs[t]))
            sh = []
            for t in range(NS):
                gh[t].wait()
                sh.append(pltpu.async_copy(rows[t], acc.at[didx[t]], ss[t], add=True))
            for t in range(NS):
                sh[t].wait()
            return carry

        lax.fori_loop(0, NCHUNK // NS, body, 0)
        plsc.subcore_barrier()
        pltpu.sync_copy(acc.at[pl.ds(s * zr, zr)],
                        out_hbm.at[pl.ds(c * ACC_ROWS + s * zr, zr)])

    return k(p, src, dst, zrows)


def _gin_tail(s, w1_ref, b1_ref, w2_ref, b2_ref, g_ref, be_ref):
    z = jnp.maximum(jnp.dot(s, w1_ref[...],
                            preferred_element_type=jnp.float32) + b1_ref[...], 0.0)
    t = jnp.dot(z, w2_ref[...], preferred_element_type=jnp.float32) + b2_ref[...]
    hh = jnp.maximum(t, 0.0)
    mu = jnp.mean(hh, axis=0, keepdims=True)
    var = jnp.mean((hh - mu) * (hh - mu), axis=0, keepdims=True)
    return (hh - mu) / jnp.sqrt(var + 1e-5) * g_ref[...] + be_ref[...]


def _gin1(xp, aggx, w1p, b1, w2, b2, g, be):
    def body(x_ref, agg_ref, w1_ref, b1_ref, w2_ref, b2_ref, g_ref, be_ref, o_ref):
        s = x_ref[...] + jnp.concatenate(
            [agg_ref[:NN], agg_ref[ACC_ROWS_X:ACC_ROWS_X + NN]], axis=1)
        o_ref[...] = _gin_tail(s, w1_ref, b1_ref, w2_ref, b2_ref, g_ref, be_ref)

    return pl.pallas_call(
        body,
        out_shape=jax.ShapeDtypeStruct((NN, DIM), jnp.float32),
    )(xp, aggx, w1p, b1, w2, b2, g, be)


def _gin_mid(h, agg, w1, b1, w2, b2, g, be):
    def body(h_ref, agg_ref, w1_ref, b1_ref, w2_ref, b2_ref, g_ref, be_ref, o_ref):
        s = h_ref[...] + agg_ref[:NN] + agg_ref[ACC_ROWS:ACC_ROWS + NN]
        o_ref[...] = _gin_tail(s, w1_ref, b1_ref, w2_ref, b2_ref, g_ref, be_ref)

    return pl.pallas_call(
        body,
        out_shape=jax.ShapeDtypeStruct((NN, DIM), jnp.float32),
    )(h, agg, w1, b1, w2, b2, g, be)


def _gin_last(h, agg, w1, b1, w2, b2, g, be, batch2d, wxd, bxd):
    def body(h_ref, agg_ref, w1_ref, b1_ref, w2_ref, b2_ref, g_ref, be_ref,
             bt_ref, wxd_ref, bxd_ref, o_ref):
        s = h_ref[...] + agg_ref[:NN] + agg_ref[ACC_ROWS:ACC_ROWS + NN]
        hn = _gin_tail(s, w1_ref, b1_ref, w2_ref, b2_ref, g_ref, be_ref)
        seg = jax.lax.broadcasted_iota(jnp.int32, (NN, NB), 1)
        onehot = (bt_ref[...] == seg).astype(jnp.float32)
        pooled = jax.lax.dot_general(onehot, hn, (((0,), (0,)), ((), ())),
                                     preferred_element_type=jnp.float32,
                                     precision=jax.lax.Precision.HIGHEST)
        o_ref[...] = jnp.maximum(
            jnp.dot(pooled, wxd_ref[...], preferred_element_type=jnp.float32)
            + bxd_ref[...], 0.0)

    return pl.pallas_call(
        body,
        out_shape=jax.ShapeDtypeStruct((NB, 128), jnp.float32),
    )(h, agg, w1, b1, w2, b2, g, be, batch2d, wxd, bxd)


def _cnn(xc, w1c, b1c, w2f, b2c, w3f, b3c):
    T1, T2, T3 = 13125, 4368, 1449
    P1, P2, P3 = 4375, 1456, 483

    def max3(ref):
        return jnp.maximum(jnp.maximum(ref[0::3, :], ref[1::3, :]), ref[2::3, :])

    def body(x_ref, w1_ref, b1_ref, w2_ref, b2_ref, w3_ref, b3_ref, o_ref,
             s1, s2, s3):
        xb = x_ref[0]
        u1t = jnp.concatenate([xb[:, kk:kk + T1] for kk in range(8)], axis=0)
        u1 = jnp.transpose(u1t)
        o1 = jnp.dot(u1, w1_ref[...], preferred_element_type=jnp.float32)
        s1[...] = jnp.maximum(o1 + b1_ref[...], 0.0)
        h1 = max3(s1)

        u2 = jnp.concatenate([h1[kk:kk + T2, :] for kk in range(8)], axis=1)
        o2 = jnp.dot(u2, w2_ref[...], preferred_element_type=jnp.float32)
        s2[...] = jnp.maximum(o2 + b2_ref[...], 0.0)
        h2 = max3(s2)

        u3 = jnp.concatenate([h2[kk:kk + T3, :] for kk in range(8)], axis=1)
        o3 = jnp.dot(u3, w3_ref[...], preferred_element_type=jnp.float32)
        s3[...] = jnp.maximum(o3 + b3_ref[...], 0.0)
        h3 = max3(s3)

        o_ref[...] = jnp.transpose(h3).reshape(1, 128, P3)

    out = pl.pallas_call(
        body,
        grid=(NB,),
        scratch_shapes=[pltpu.VMEM((T1, 32), jnp.float32),
                        pltpu.VMEM((T2, 64), jnp.float32),
                        pltpu.VMEM((T3, 128), jnp.float32)],
        in_specs=[pl.BlockSpec((1, 1, 13132), lambda i: (i, 0, 0)),
                  pl.BlockSpec((8, 32), lambda i: (0, 0)),
                  pl.BlockSpec((1, 32), lambda i: (0, 0)),
                  pl.BlockSpec((256, 64), lambda i: (0, 0)),
                  pl.BlockSpec((1, 64), lambda i: (0, 0)),
                  pl.BlockSpec((512, 128), lambda i: (0, 0)),
                  pl.BlockSpec((1, 128), lambda i: (0, 0))],
        out_specs=pl.BlockSpec((1, 128, P3), lambda i: (i, 0, 0)),
        out_shape=jax.ShapeDtypeStruct((NB, 128, P3), jnp.float32),
    )(xc.reshape(NB, 1, 13132), w1c, b1c, w2f, b2c, w3f, b3c)
    return out.reshape(NB, 61824)


def _fc_xt(ct, w, b):
    KB = 8832

    def body(a_ref, w_ref, b_ref, o_ref):
        @pl.when(pl.program_id(0) == 0)
        def _():
            o_ref[...] = jnp.broadcast_to(b_ref[...], (NB, 128))
        o_ref[...] += jnp.dot(a_ref[...], w_ref[...],
                              preferred_element_type=jnp.float32)

    return pl.pallas_call(
        body,
        grid=(61824 // KB,),
        in_specs=[pl.BlockSpec((NB, KB), lambda k: (0, k)),
                  pl.BlockSpec((KB, 128), lambda k: (k, 0)),
                  pl.BlockSpec((1, 128), lambda k: (0, 0))],
        out_specs=pl.BlockSpec((NB, 128), lambda k: (0, 0)),
        out_shape=jax.ShapeDtypeStruct((NB, 128), jnp.float32),
    )(ct, w, b)


def _head(xg, ct, w1, b1, w2, b2, wo, bo):
    def body(xg_ref, ct_ref, w1_ref, b1_ref, w2_ref, b2_ref, wo_ref, bo_ref, o_ref):
        xc = jnp.concatenate([xg_ref[...], ct_ref[...]], axis=1)
        h = jnp.maximum(jnp.dot(xc, w1_ref[...],
                                preferred_element_type=jnp.float32) + b1_ref[...], 0.0)
        h = jnp.maximum(jnp.dot(h, w2_ref[...],
                                preferred_element_type=jnp.float32) + b2_ref[...], 0.0)
        o_ref[...] = jax.nn.sigmoid(
            jnp.dot(h, wo_ref[...], preferred_element_type=jnp.float32) + bo_ref[...])

    return pl.pallas_call(
        body,
        out_shape=jax.ShapeDtypeStruct((NB, 1), jnp.float32),
    )(xg, ct, w1, b1, w2, b2, wo, bo)


def kernel(x, edge_index, batch, x_cell_mut, edge_feat, params):
    gin = params['gin']
    bn = params['bn']

    src = edge_index[0].astype(jnp.int32)
    dst = edge_index[1].astype(jnp.int32)
    npad = EPAD - E
    src_p = jnp.concatenate([src, jnp.zeros((npad,), jnp.int32)])
    dst_p = jnp.concatenate([dst, jnp.full((npad,), NN, jnp.int32)])
    src2 = jnp.concatenate([src_p, src_p + NN])
    dst2 = jnp.concatenate([dst_p, dst_p])
    zrows = jnp.zeros((ACC_ROWS, DIM), jnp.float32)
    z192 = jnp.zeros((ACC_ROWS_X, FH), jnp.float32)

    def r2(v):
        return v.reshape(1, -1)

    xp = jnp.concatenate([x, jnp.zeros((NN, FPAD - x.shape[1]), jnp.float32)], axis=1)
    xf = jnp.concatenate([xp[:, :FH], xp[:, FH:]], axis=0)
    w1p = jnp.concatenate(
        [gin[0][0], jnp.zeros((FPAD - x.shape[1], DIM), jnp.float32)], axis=0)
    aggx = _edge_agg_x(xf, src2, dst2, z192)
    (w1, b1, w2, b2) = gin[0]
    (g, be) = bn[0]
    h = _gin1(xp, aggx, w1p, r2(b1), w2, r2(b2), r2(g), r2(be))
    for l in range(1, 4):
        agg = _edge_agg(h, src_p, dst_p, zrows)
        (w1, b1, w2, b2) = gin[l]
        (g, be) = bn[l]
        h = _gin_mid(h, agg, w1, r2(b1), w2, r2(b2), r2(g), r2(be))
    agg = _edge_agg(h, src_p, dst_p, zrows)
    (w1, b1, w2, b2) = gin[4]
    (g, be) = bn[4]
    wxd, bxd = params['fc1_xd']
    xg = _gin_last(h, agg, w1, r2(b1), w2, r2(b2), r2(g), r2(be),
                   batch.reshape(NN, 1).astype(jnp.int32), wxd, r2(bxd))

    (cw1, cb1), (cw2, cb2), (cw3, cb3) = params['conv_xt']
    w1c = cw1.reshape(32, 8).T
    w2f = cw2.transpose(2, 1, 0).reshape(256, 64)
    w3f = cw3.transpose(2, 1, 0).reshape(512, 128)
    ctf = _cnn(x_cell_mut.reshape(NB, 13132), w1c, cb1.reshape(1, 32),
               w2f, cb2.reshape(1, 64), w3f, cb3.reshape(1, 128))
    wxt, bxt = params['fc1_xt']
    ct = _fc_xt(ctf, wxt, r2(bxt))

    w1h, b1h = params['fc1']
    w2h, b2h = params['fc2']
    wo, bo = params['out']
    return _head(xg, ct, w1h, r2(b1h), w2h, r2(b2h), wo, r2(bo))


# speedup vs baseline: 2.8880x; 1.0135x over previous
import functools

import jax
import jax.numpy as jnp
from jax import lax
from jax.experimental import pallas as pl
from jax.experimental.pallas import tpu as pltpu
from jax.experimental.pallas import tpu_sc as plsc

NN = 10000
NB = 32
DIM = 32
E = 160000
EPAD = 163840
CH = 128
NCHUNK = EPAD // (32 * CH)
ACC_ROWS = 10240


FPAD = 352
FH = FPAD // 2
ACC_ROWS_X = 10112
CHX = 40


def _edge_agg_x(xf, src2, dst2, z192):
    mesh = plsc.VectorSubcoreMesh(core_axis_name="c", subcore_axis_name="s")
    epr = EPAD // 16
    nch = epr // CHX
    sl = ACC_ROWS_X // 16

    @functools.partial(
        pl.kernel,
        out_type=jax.ShapeDtypeStruct((2 * ACC_ROWS_X, FH), jnp.float32),
        mesh=mesh,
        compiler_params=pltpu.CompilerParams(use_tc_tiling_on_sc=False),
        scratch_types=[
            pltpu.VMEM_SHARED((ACC_ROWS_X, FH), jnp.float32),
            pltpu.VMEM((CHX,), jnp.int32), pltpu.VMEM((CHX,), jnp.int32),
            pltpu.VMEM((CHX,), jnp.int32), pltpu.VMEM((CHX,), jnp.int32),
            pltpu.VMEM((CHX, FH), jnp.float32), pltpu.VMEM((CHX, FH), jnp.float32),
            pltpu.SemaphoreType.DMA, pltpu.SemaphoreType.DMA,
            pltpu.SemaphoreType.DMA, pltpu.SemaphoreType.DMA,
        ],
    )
    def k(x_hbm, src_hbm, dst_hbm, z_hbm, out_hbm, acc,
          sidx0, sidx1, didx0, didx1, rows0, rows1, gs0, gs1, ss0, ss1):
        c = lax.axis_index("c")
        s = lax.axis_index("s")
        pltpu.sync_copy(z_hbm.at[pl.ds(s * sl, sl)], acc.at[pl.ds(s * sl, sl)])
        plsc.subcore_barrier()

        ebase = c * EPAD + s * epr

        def body(i, carry):
            c0 = ebase + (2 * i) * CHX
            c1 = c0 + CHX
            pltpu.sync_copy(src_hbm.at[pl.ds(c0, CHX)], sidx0)
            pltpu.sync_copy(dst_hbm.at[pl.ds(c0, CHX)], didx0)
            g0 = pltpu.async_copy(x_hbm.at[sidx0], rows0, gs0)
            pltpu.sync_copy(src_hbm.at[pl.ds(c1, CHX)], sidx1)
            pltpu.sync_copy(dst_hbm.at[pl.ds(c1, CHX)], didx1)
            g1 = pltpu.async_copy(x_hbm.at[sidx1], rows1, gs1)
            g0.wait()
            s0 = pltpu.async_copy(rows0, acc.at[didx0], ss0, add=True)
            g1.wait()
            s1 = pltpu.async_copy(rows1, acc.at[didx1], ss1, add=True)
            s0.wait()
            s1.wait()
            return carry

        lax.fori_loop(0, nch // 2, body, 0)
        plsc.subcore_barrier()
        pltpu.sync_copy(acc.at[pl.ds(s * sl, sl)],
                        out_hbm.at[pl.ds(c * ACC_ROWS_X + s * sl, sl)])

    return k(xf, src2, dst2, z192)


def _edge_agg(p, src, dst, zrows):
    mesh = plsc.VectorSubcoreMesh(core_axis_name="c", subcore_axis_name="s")

    NS = 8

    @functools.partial(
        pl.kernel,
        out_type=jax.ShapeDtypeStruct((2 * ACC_ROWS, DIM), jnp.float32),
        mesh=mesh,
        compiler_params=pltpu.CompilerParams(use_tc_tiling_on_sc=False),
        scratch_types=[
            pltpu.VMEM_SHARED((ACC_ROWS, DIM), jnp.float32),
            [pltpu.VMEM((CH,), jnp.int32)] * NS,
            [pltpu.VMEM((CH,), jnp.int32)] * NS,
            [pltpu.VMEM((CH, DIM), jnp.float32)] * NS,
            [pltpu.SemaphoreType.DMA] * NS,
            [pltpu.SemaphoreType.DMA] * NS,
        ],
    )
    def k(p_hbm, src_hbm, dst_hbm, z_hbm, out_hbm, acc, sidx, didx, rows, gs, ss):
        c = lax.axis_index("c")
        s = lax.axis_index("s")
        wid = c * 16 + s
        zr = ACC_ROWS // 16
        pltpu.sync_copy(z_hbm.at[pl.ds(s * zr, zr)], acc.at[pl.ds(s * zr, zr)])
        plsc.subcore_barrier()

        ebase = wid * (NCHUNK * CH)

        def body(i, carry):
            base = ebase + (NS * i) * CH
            gh = []
            for t in range(NS):
                off = base + t * CH
                pltpu.sync_copy(src_hbm.at[pl.ds(off, CH)], sidx[t])
                pltpu.sync_copy(dst_hbm.at[pl.ds(off, CH)], didx[t])
                gh.append(pltpu.async_copy(p_hbm.at[sidx[t]], rows[t], gs[t]))
            sh = []
            for t in range(NS):
                gh[t].wait()
                sh.append(pltpu.async_copy(rows[t], acc.at[didx[t]], ss[t], add=True))
            for t in range(NS):
                sh[t].wait()
            return carry

        lax.fori_loop(0, NCHUNK // NS, body, 0)
        plsc.subcore_barrier()
        pltpu.sync_copy(acc.at[pl.ds(s * zr, zr)],
                        out_hbm.at[pl.ds(c * ACC_ROWS + s * zr, zr)])

    return k(p, src, dst, zrows)


def _gin_tail(s, w1_ref, b1_ref, w2_ref, b2_ref, g_ref, be_ref):
    z = jnp.maximum(jnp.dot(s, w1_ref[...],
                            preferred_element_type=jnp.float32) + b1_ref[...], 0.0)
    t = jnp.dot(z, w2_ref[...], preferred_element_type=jnp.float32) + b2_ref[...]
    hh = jnp.maximum(t, 0.0)
    mu = jnp.mean(hh, axis=0, keepdims=True)
    var = jnp.mean((hh - mu) * (hh - mu), axis=0, keepdims=True)
    return (hh - mu) / jnp.sqrt(var + 1e-5) * g_ref[...] + be_ref[...]


def _gin1(xp, aggx, w1p, b1, w2, b2, g, be):
    def body(x_ref, agg_ref, w1_ref, b1_ref, w2_ref, b2_ref, g_ref, be_ref, o_ref):
        s = x_ref[...] + jnp.concatenate(
            [agg_ref[:NN], agg_ref[ACC_ROWS_X:ACC_ROWS_X + NN]], axis=1)
        o_ref[...] = _gin_tail(s, w1_ref, b1_ref, w2_ref, b2_ref, g_ref, be_ref)

    return pl.pallas_call(
        body,
        out_shape=jax.ShapeDtypeStruct((NN, DIM), jnp.float32),
    )(xp, aggx, w1p, b1, w2, b2, g, be)


def _gin_mid(h, agg, w1, b1, w2, b2, g, be):
    def body(h_ref, agg_ref, w1_ref, b1_ref, w2_ref, b2_ref, g_ref, be_ref, o_ref):
        s = h_ref[...] + agg_ref[:NN] + agg_ref[ACC_ROWS:ACC_ROWS + NN]
        o_ref[...] = _gin_tail(s, w1_ref, b1_ref, w2_ref, b2_ref, g_ref, be_ref)

    return pl.pallas_call(
        body,
        out_shape=jax.ShapeDtypeStruct((NN, DIM), jnp.float32),
    )(h, agg, w1, b1, w2, b2, g, be)


def _gin_last(h, agg, w1, b1, w2, b2, g, be, batch2d, wxd, bxd):
    def body(h_ref, agg_ref, w1_ref, b1_ref, w2_ref, b2_ref, g_ref, be_ref,
             bt_ref, wxd_ref, bxd_ref, o_ref):
        s = h_ref[...] + agg_ref[:NN] + agg_ref[ACC_ROWS:ACC_ROWS + NN]
        hn = _gin_tail(s, w1_ref, b1_ref, w2_ref, b2_ref, g_ref, be_ref)
        seg = jax.lax.broadcasted_iota(jnp.int32, (NN, NB), 1)
        onehot = (bt_ref[...] == seg).astype(jnp.float32)
        pooled = jax.lax.dot_general(onehot, hn, (((0,), (0,)), ((), ())),
                                     preferred_element_type=jnp.float32,
                                     precision=jax.lax.Precision.HIGHEST)
        o_ref[...] = jnp.maximum(
            jnp.dot(pooled, wxd_ref[...], preferred_element_type=jnp.float32)
            + bxd_ref[...], 0.0)

    return pl.pallas_call(
        body,
        out_shape=jax.ShapeDtypeStruct((NB, 128), jnp.float32),
    )(h, agg, w1, b1, w2, b2, g, be, batch2d, wxd, bxd)


def _cnn(xc, w1c, b1c, w2f, b2c, w3f, b3c):
    T1, T2, T3 = 13125, 4368, 1449
    P1, P2, P3 = 4375, 1456, 483

    def max3(ref):
        return jnp.maximum(jnp.maximum(ref[0::3, :], ref[1::3, :]), ref[2::3, :])

    def body(x_ref, w1_ref, b1_ref, w2_ref, b2_ref, w3_ref, b3_ref, o_ref,
             s1, s2, s3):
        xb = x_ref[0]
        u1t = jnp.concatenate([xb[:, kk:kk + T1] for kk in range(8)], axis=0)
        u1 = jnp.transpose(u1t)
        o1 = jnp.dot(u1, w1_ref[...], preferred_element_type=jnp.float32)
        s1[...] = jnp.maximum(o1 + b1_ref[...], 0.0)
        h1 = max3(s1)

        u2 = jnp.concatenate([h1[kk:kk + T2, :] for kk in range(8)], axis=1)
        o2 = jnp.dot(u2, w2_ref[...], preferred_element_type=jnp.float32)
        s2[...] = jnp.maximum(o2 + b2_ref[...], 0.0)
        h2 = max3(s2)

        u3 = jnp.concatenate([h2[kk:kk + T3, :] for kk in range(8)], axis=1)
        o3 = jnp.dot(u3, w3_ref[...], preferred_element_type=jnp.float32)
        s3[...] = jnp.maximum(o3 + b3_ref[...], 0.0)
        h3 = max3(s3)

        o_ref[...] = jnp.transpose(h3).reshape(1, 128, P3)

    out = pl.pallas_call(
        body,
        grid=(NB,),
        scratch_shapes=[pltpu.VMEM((T1, 32), jnp.float32),
                        pltpu.VMEM((T2, 64), jnp.float32),
                        pltpu.VMEM((T3, 128), jnp.float32)],
        in_specs=[pl.BlockSpec((1, 1, 13132), lambda i: (i, 0, 0)),
                  pl.BlockSpec((8, 32), lambda i: (0, 0)),
                  pl.BlockSpec((1, 32), lambda i: (0, 0)),
                  pl.BlockSpec((256, 64), lambda i: (0, 0)),
                  pl.BlockSpec((1, 64), lambda i: (0, 0)),
                  pl.BlockSpec((512, 128), lambda i: (0, 0)),
                  pl.BlockSpec((1, 128), lambda i: (0, 0))],
        out_specs=pl.BlockSpec((1, 128, P3), lambda i: (i, 0, 0)),
        out_shape=jax.ShapeDtypeStruct((NB, 128, P3), jnp.float32),
    )(xc.reshape(NB, 1, 13132), w1c, b1c, w2f, b2c, w3f, b3c)
    return out.reshape(NB, 61824)


def _fc_xt(ct, w, b):
    KB = 8832

    def body(a_ref, w_ref, b_ref, o_ref):
        @pl.when(pl.program_id(0) == 0)
        def _():
            o_ref[...] = jnp.broadcast_to(b_ref[...], (NB, 128))
        o_ref[...] += jnp.dot(a_ref[...], w_ref[...],
                              preferred_element_type=jnp.float32)

    return pl.pallas_call(
        body,
        grid=(61824 // KB,),
        in_specs=[pl.BlockSpec((NB, KB), lambda k: (0, k)),
                  pl.BlockSpec((KB, 128), lambda k: (k, 0)),
                  pl.BlockSpec((1, 128), lambda k: (0, 0))],
        out_specs=pl.BlockSpec((NB, 128), lambda k: (0, 0)),
        out_shape=jax.ShapeDtypeStruct((NB, 128), jnp.float32),
    )(ct, w, b)


def _head(xg, ct, w1, b1, w2, b2, wo, bo):
    def body(xg_ref, ct_ref, w1_ref, b1_ref, w2_ref, b2_ref, wo_ref, bo_ref, o_ref):
        xc = jnp.concatenate([xg_ref[...], ct_ref[...]], axis=1)
        h = jnp.maximum(jnp.dot(xc, w1_ref[...],
                                preferred_element_type=jnp.float32) + b1_ref[...], 0.0)
        h = jnp.maximum(jnp.dot(h, w2_ref[...],
                                preferred_element_type=jnp.float32) + b2_ref[...], 0.0)
        o_ref[...] = jax.nn.sigmoid(
            jnp.dot(h, wo_ref[...], preferred_element_type=jnp.float32) + bo_ref[...])

    return pl.pallas_call(
        body,
        out_shape=jax.ShapeDtypeStruct((NB, 1), jnp.float32),
    )(xg, ct, w1, b1, w2, b2, wo, bo)


def kernel(x, edge_index, batch, x_cell_mut, edge_feat, params):
    gin = params['gin']
    bn = params['bn']

    src = edge_index[0].astype(jnp.int32)
    dst = edge_index[1].astype(jnp.int32)
    npad = EPAD - E
    src_p = jnp.concatenate([src, jnp.zeros((npad,), jnp.int32)])
    dst_p = jnp.concatenate([dst, jnp.full((npad,), NN, jnp.int32)])
    src2 = jnp.concatenate([src_p, src_p + NN])
    dst2 = jnp.concatenate([dst_p, dst_p])
    zrows = jnp.zeros((ACC_ROWS, DIM), jnp.float32)
    z192 = jnp.zeros((ACC_ROWS_X, FH), jnp.float32)

    def r2(v):
        return v.reshape(1, -1)

    xp = jnp.concatenate([x, jnp.zeros((NN, FPAD - x.shape[1]), jnp.float32)], axis=1)
    xf = jnp.concatenate([xp[:, :FH], xp[:, FH:]], axis=0)
    w1p = jnp.concatenate(
        [gin[0][0], jnp.zeros((FPAD - x.shape[1], DIM), jnp.float32)], axis=0)
    aggx = _edge_agg_x(xf, src2, dst2, z192)
    (w1, b1, w2, b2) = gin[0]
    (g, be) = bn[0]
    h = _gin1(xp, aggx, w1p, r2(b1), w2, r2(b2), r2(g), r2(be))
    for l in range(1, 4):
        agg = _edge_agg(h, src_p, dst_p, zrows)
        (w1, b1, w2, b2) = gin[l]
        (g, be) = bn[l]
        h = _gin_mid(h, agg, w1, r2(b1), w2, r2(b2), r2(g), r2(be))
    agg = _edge_agg(h, src_p, dst_p, zrows)
    (w1, b1, w2, b2) = gin[4]
    (g, be) = bn[4]
    wxd, bxd = params['fc1_xd']
    xg = _gin_last(h, agg, w1, r2(b1), w2, r2(b2), r2(g), r2(be),
                   batch.reshape(NN, 1).astype(jnp.int32), wxd, r2(bxd))

    (cw1, cb1), (cw2, cb2), (cw3, cb3) = params['conv_xt']
    w1c = cw1.reshape(32, 8).T
    w2f = cw2.transpose(2, 1, 0).reshape(256, 64)
    w3f = cw3.transpose(2, 1, 0).reshape(512, 128)
    ctf = _cnn(x_cell_mut.reshape(NB, 13132), w1c, cb1.reshape(1, 32),
               w2f, cb2.reshape(1, 64), w3f, cb3.reshape(1, 128))
    wxt, bxt = params['fc1_xt']
    ct = _fc_xt(ctf, wxt, r2(bxt))

    w1h, b1h = params['fc1']
    w2h, b2h = params['fc2']
    wo, bo = params['out']
    return _head(xg, ct, w1h, r2(b1h), w2h, r2(b2h), wo, r2(bo))
